# Initial kernel scaffold; baseline (speedup 1.0000x reference)
import functools
from math import log

import jax
import jax.numpy as jnp
from jax import lax
from jax.experimental import pallas as pl
from jax.experimental.pallas import tpu as pltpu
from jax.experimental.pallas import tpu_sc as plsc

_B, _N, _K, _C = 2, 5000, 16, 256
_ALPHA = 0.1
_BETA = log(0.5 / 4.0 + 1.0)
_C1 = (1.0 - _ALPHA) * (1.0 - _BETA)
_C2 = _ALPHA * (1.0 - _BETA)

_BN = _B * _N
_NW = 32
_PW = 320
_BNP = _NW * _PW
_G = 8
_GK = _G * _K
_NCH = _PW // _G
_LC = _C // 16


def _sc_gather_sum(xf_hbm, idx_hbm, out_hbm, idx_v, rows_v, self_v, acc_v,
                   sem, sem2):
    nc = 2
    wid = lax.axis_index("s") * nc + lax.axis_index("c")
    node0 = wid * _PW
    pltpu.sync_copy(idx_hbm.at[pl.ds(wid * _NCH, _NCH)], idx_v)

    def chunk(t, _):
        nb = node0 + t * _G
        cp = pltpu.async_copy(xf_hbm.at[idx_v.at[t]], rows_v, sem)
        cp2 = pltpu.async_copy(xf_hbm.at[pl.ds(nb, _G)], self_v, sem2)
        cp.wait()
        cp2.wait()

        def per_g(g, _):
            def per_c(c, _):
                v = self_v[g, pl.ds(c * 16, 16)]
                for k in range(_K):
                    v = v + rows_v[g * _K + k, pl.ds(c * 16, 16)]
                acc_v[g, pl.ds(c * 16, 16)] = v
                return 0

            return lax.fori_loop(0, _LC, per_c, 0)

        lax.fori_loop(0, _G, per_g, 0)
        pltpu.sync_copy(acc_v, out_hbm.at[pl.ds(nb, _G)])
        return 0

    lax.fori_loop(0, _NCH, chunk, 0)


_sc_call = functools.partial(
    pl.kernel,
    out_type=jax.ShapeDtypeStruct((_BNP, _C), jnp.float32),
    mesh=plsc.VectorSubcoreMesh(core_axis_name="c", subcore_axis_name="s"),
    scratch_types=[
        pltpu.VMEM((_NCH, _GK), jnp.int32),
        pltpu.VMEM((_GK, _C), jnp.float32),
        pltpu.VMEM((_G, _C), jnp.float32),
        pltpu.VMEM((_G, _C), jnp.float32),
        pltpu.SemaphoreType.DMA,
        pltpu.SemaphoreType.DMA,
    ],
)


_NB = 1000


def _tc_transpose_body(x_ref, out_ref):
    out_ref[...] = x_ref[0].T


def _tc_combine_body(aggr_ref, x0_ref, w1t_ref, w2t_ref, p_ref, out_ref):
    a = aggr_ref[...]
    z = x0_ref[...]
    h = (a * _C1 + z * _C2
         + jnp.dot(a, w1t_ref[...], preferred_element_type=jnp.float32) * _BETA
         + jnp.dot(z, w2t_ref[...], preferred_element_type=jnp.float32) * _BETA
         + p_ref[0:1, :])
    h = jnp.maximum(h, 0.0)
    mean = jnp.mean(h, axis=1, keepdims=True)
    d = h - mean
    var = jnp.mean(d * d, axis=1, keepdims=True)
    y = d * lax.rsqrt(var + 1e-5) * p_ref[1:2, :] + p_ref[2:3, :]
    out_ref[0] = y.T


def kernel(x, x_0, edge_index, W1, W2, bias, ln_gamma, ln_beta):
    f32 = jnp.float32
    xt = x.reshape(_B, _C, _N)

    xf = pl.pallas_call(
        _tc_transpose_body,
        grid=(_B, _N // _NB),
        in_specs=[pl.BlockSpec((1, _C, _NB), lambda b, j: (b, 0, j))],
        out_specs=pl.BlockSpec((_NB, _C), lambda b, j: (b * (_N // _NB) + j, 0)),
        out_shape=jax.ShapeDtypeStruct((_BNP, _C), f32),
        compiler_params=pltpu.CompilerParams(
            dimension_semantics=("parallel", "parallel")),
    )(xt)

    idx = edge_index[0] + (jnp.arange(_B, dtype=jnp.int32) * _N).reshape(_B, 1, 1)
    idx = idx.reshape(_BN * _K)
    idx = jnp.concatenate(
        [idx, jnp.zeros(((_BNP - _BN) * _K,), jnp.int32)]).reshape(
            _BNP // _G, _GK)

    aggr = _sc_call(_sc_gather_sum)(xf, idx)

    x0f = x_0.reshape(_BN, _C)
    params = jnp.concatenate([bias.reshape(1, _C),
                              ln_gamma.reshape(1, _C),
                              ln_beta.reshape(1, _C)], axis=0)
    njb = _N // _NB
    out = pl.pallas_call(
        _tc_combine_body,
        grid=(_B, njb),
        in_specs=[
            pl.BlockSpec((_NB, _C), lambda b, j: (b * njb + j, 0)),
            pl.BlockSpec((_NB, _C), lambda b, j: (b * njb + j, 0)),
            pl.BlockSpec((_C, _C), lambda b, j: (0, 0)),
            pl.BlockSpec((_C, _C), lambda b, j: (0, 0)),
            pl.BlockSpec((3, _C), lambda b, j: (0, 0)),
        ],
        out_specs=pl.BlockSpec((1, _C, _NB), lambda b, j: (b, 0, j)),
        out_shape=jax.ShapeDtypeStruct((_B, _C, _N), f32),
        compiler_params=pltpu.CompilerParams(
            dimension_semantics=("parallel", "parallel")),
    )(aggr, x0f, W1.T, W2.T, params)

    return out[..., None]


# speedup vs baseline: 4.2840x; 4.2840x over previous
import functools
from math import log

import jax
import jax.numpy as jnp
from jax import lax
from jax.experimental import pallas as pl
from jax.experimental.pallas import tpu as pltpu
from jax.experimental.pallas import tpu_sc as plsc

_B, _N, _K, _C = 2, 5000, 16, 256
_ALPHA = 0.1
_BETA = log(0.5 / 4.0 + 1.0)
_C1 = (1.0 - _ALPHA) * (1.0 - _BETA)
_C2 = _ALPHA * (1.0 - _BETA)

_BN = _B * _N
_G = 8
_GK = _G * _K
_NCHT = _BN // _G
_WCH = 40
_SWIN = 64
_NPAIR = _WCH // 2
_CW = _C // 2
_BCH = (_N // _G)


def _sc_gather_sum(xf_hbm, eidx_hbm, out_hbm, idx_v, rows_a, rows_b,
                   acc_a, acc_b, sem_i, sem_a, sem_b, sem_sa, sem_sb):
    nc = 2
    wid = lax.axis_index("s") * nc + lax.axis_index("c")
    p = (wid * (_NCHT // 2)) // 16
    sa = (p // 8) * 8
    lo = p - sa
    pltpu.async_copy(eidx_hbm.at[pl.ds(sa, _SWIN)], idx_v, sem_i).wait()

    def fix(t, _):
        off = jnp.where(sa + t >= _BCH, _N, 0).astype(jnp.int32)
        for j in range(_GK // 16):
            idx_v[t, pl.ds(j * 16, 16)] = idx_v[t, pl.ds(j * 16, 16)] + off
        return 0

    lax.fori_loop(0, _SWIN, fix, 0)

    def gather(t, rows, sem):
        return pltpu.async_copy(xf_hbm.at[idx_v.at[lo + t]], rows, sem)

    def compute(rows, acc):
        def per_g(g, _):
            base = g * _K
            for j in range(_CW // 16):
                sl = pl.ds(j * 16, 16)
                vs = [plsc.bitcast(rows[base + k, sl], jnp.bfloat16)
                      for k in range(_K)]
                while len(vs) > 1:
                    vs = [a + b for a, b in zip(vs[::2], vs[1::2])]
                acc[g, sl] = plsc.bitcast(vs[0], jnp.float32)
            return 0

        lax.fori_loop(0, _G, per_g, 0)

    def store(t, acc, sem):
        pltpu.async_copy(acc, out_hbm.at[pl.ds((p + t) * _G, _G)], sem)

    def store_wait(t, acc, sem):
        pltpu.make_async_copy(acc, out_hbm.at[pl.ds((p + t) * _G, _G)],
                              sem).wait()

    gather(0, rows_a, sem_a)

    def pair(tt, _):
        t0 = 2 * tt
        gather(t0 + 1, rows_b, sem_b)
        pltpu.make_async_copy(xf_hbm.at[idx_v.at[lo + t0]], rows_a,
                              sem_a).wait()

        @pl.when(tt > 0)
        def _():
            store_wait(t0 - 2, acc_a, sem_sa)

        compute(rows_a, acc_a)
        store(t0, acc_a, sem_sa)

        @pl.when(tt < _NPAIR - 1)
        def _():
            gather(t0 + 2, rows_a, sem_a)

        pltpu.make_async_copy(xf_hbm.at[idx_v.at[lo + t0 + 1]], rows_b,
                              sem_b).wait()

        @pl.when(tt > 0)
        def _():
            store_wait(t0 - 1, acc_b, sem_sb)

        compute(rows_b, acc_b)
        store(t0 + 1, acc_b, sem_sb)
        return 0

    lax.fori_loop(0, _NPAIR, pair, 0)
    store_wait(_WCH - 2, acc_a, sem_sa)
    store_wait(_WCH - 1, acc_b, sem_sb)


def _sc_call():
    return functools.partial(
        pl.kernel,
        out_type=jax.ShapeDtypeStruct((_BN, _CW), jnp.float32),
        mesh=plsc.VectorSubcoreMesh(core_axis_name="c", subcore_axis_name="s",
                                    num_cores=2, num_subcores=16),
        compiler_params=pltpu.CompilerParams(needs_layout_passes=False),
        scratch_types=[
            pltpu.VMEM((_SWIN, _GK), jnp.int32),
            pltpu.VMEM((_GK, _CW), jnp.float32),
            pltpu.VMEM((_GK, _CW), jnp.float32),
            pltpu.VMEM((_G, _CW), jnp.float32),
            pltpu.VMEM((_G, _CW), jnp.float32),
            pltpu.SemaphoreType.DMA,
            pltpu.SemaphoreType.DMA,
            pltpu.SemaphoreType.DMA,
            pltpu.SemaphoreType.DMA,
            pltpu.SemaphoreType.DMA,
        ],
    )


def _tc_transpose_body(x_ref, out_ref):
    y = x_ref[0].astype(jnp.bfloat16)
    out_ref[...] = pltpu.bitcast(y, jnp.float32).T


def _tc_combine_body(aggr_ref, xf_ref, x0_ref, w1_ref, w2_ref, p_ref,
                     out_ref):
    def unpack(ref):
        return pltpu.bitcast(ref[...].T, jnp.bfloat16).astype(jnp.float32)

    a = unpack(aggr_ref) + unpack(xf_ref)
    h = (a * _C1
         + jnp.dot(w1_ref[...], a.astype(jnp.bfloat16),
                   preferred_element_type=jnp.float32)
         + lax.dot_general(w2_ref[...], x0_ref[...].astype(jnp.bfloat16),
                           (((1,), (1,)), ((), ())),
                           preferred_element_type=jnp.float32)
         + p_ref[0:1, :].T)
    h = jnp.maximum(h, 0.0)
    mean = jnp.mean(h, axis=0, keepdims=True)
    d = h - mean
    var = jnp.mean(d * d, axis=0, keepdims=True)
    y = d * lax.rsqrt(var + 1e-5) * p_ref[1:2, :].T + p_ref[2:3, :].T
    out_ref[0] = y


def kernel(x, x_0, edge_index, W1, W2, bias, ln_gamma, ln_beta):
    f32 = jnp.float32
    xt = x.reshape(_B, _C, _N)

    xf = pl.pallas_call(
        _tc_transpose_body,
        grid=(_B,),
        in_specs=[pl.BlockSpec((1, _C, _N), lambda b: (b, 0, 0))],
        out_specs=pl.BlockSpec((_N, _CW), lambda b: (b, 0)),
        out_shape=jax.ShapeDtypeStruct((_BN, _CW), jnp.float32),
        compiler_params=pltpu.CompilerParams(
            dimension_semantics=("parallel",)),
    )(xt)

    eidx = edge_index.reshape(2 * _NCHT, _GK)
    aggr = _sc_call()(_sc_gather_sum)(xf, eidx)

    x0f = x_0.reshape(_BN, _C)
    params = jnp.concatenate([bias.reshape(1, _C),
                              ln_gamma.reshape(1, _C),
                              ln_beta.reshape(1, _C)], axis=0)
    out = pl.pallas_call(
        _tc_combine_body,
        grid=(_B,),
        in_specs=[
            pl.BlockSpec((_N, _CW), lambda b: (b, 0)),
            pl.BlockSpec((_N, _CW), lambda b: (b, 0)),
            pl.BlockSpec((_N, _C), lambda b: (b, 0)),
            pl.BlockSpec((_C, _C), lambda b: (0, 0)),
            pl.BlockSpec((_C, _C), lambda b: (0, 0)),
            pl.BlockSpec((3, _C), lambda b: (0, 0)),
        ],
        out_specs=pl.BlockSpec((1, _C, _N), lambda b: (b, 0, 0)),
        out_shape=jax.ShapeDtypeStruct((_B, _C, _N), f32),
        compiler_params=pltpu.CompilerParams(
            dimension_semantics=("parallel",)),
    )(aggr, xf, x0f,
      (W1 * _BETA).astype(jnp.bfloat16),
      (W2 * _BETA + _C2 * jnp.eye(_C, dtype=f32)).astype(jnp.bfloat16),
      params)

    return out[..., None]
